# baseline (speedup 1.0000x reference)
import jax
import jax.numpy as jnp
from jax.experimental import pallas as pl
from jax.experimental.pallas import tpu as pltpu

NUM_GROUPS = 4
TOKENS_PER_GROUP = 8192
HIDDEN = 4096
NUM_EXPERTS = 64
TILE = 1024
ROWS = TILE // 2
TILES_PER_GROUP = TOKENS_PER_GROUP // TILE


def _router_body(x_ref, w_ref, b_ref, logits_ref, probs_ref, zpart_ref):
    xp = x_ref[0]
    w = w_ref[...]
    bvec = b_ref[...]
    l0 = jnp.dot(xp[:, :HIDDEN], w,
                 preferred_element_type=jnp.float32) + bvec
    l1 = jnp.dot(xp[:, HIDDEN:], w,
                 preferred_element_type=jnp.float32) + bvec
    logits_ref[0] = jnp.concatenate([l0, l1], axis=1)
    m0 = jnp.max(l0, axis=-1, keepdims=True)
    m1 = jnp.max(l1, axis=-1, keepdims=True)
    e0 = jnp.exp(l0 - m0)
    e1 = jnp.exp(l1 - m1)
    s0 = jnp.sum(e0, axis=-1, keepdims=True)
    s1 = jnp.sum(e1, axis=-1, keepdims=True)
    probs_ref[0] = jnp.concatenate([e0 / s0, e1 / s1], axis=1)
    lz0 = m0 + jnp.log(s0)
    lz1 = m1 + jnp.log(s1)
    zpart_ref[0, 0, 0] = jnp.sum(lz0 * lz0) + jnp.sum(lz1 * lz1)


def kernel(token_inputs, W, b, expert_capacity):
    n_tokens = NUM_GROUPS * TOKENS_PER_GROUP
    n_tiles = n_tokens // TILE
    xp = token_inputs.reshape(NUM_GROUPS, TOKENS_PER_GROUP // 2, 2 * HIDDEN)
    b2 = b.reshape(1, NUM_EXPERTS)
    paired = (NUM_GROUPS, TOKENS_PER_GROUP // 2, 2 * NUM_EXPERTS)
    shape3 = (NUM_GROUPS, TOKENS_PER_GROUP, NUM_EXPERTS)
    logits, probs, zparts = pl.pallas_call(
        _router_body,
        grid=(n_tiles,),
        in_specs=[
            pl.BlockSpec((1, ROWS, 2 * HIDDEN),
                         lambda i: (i // TILES_PER_GROUP,
                                    i % TILES_PER_GROUP, 0)),
            pl.BlockSpec((HIDDEN, NUM_EXPERTS), lambda i: (0, 0)),
            pl.BlockSpec((1, NUM_EXPERTS), lambda i: (0, 0)),
        ],
        out_specs=[
            pl.BlockSpec((1, ROWS, 2 * NUM_EXPERTS),
                         lambda i: (i // TILES_PER_GROUP,
                                    i % TILES_PER_GROUP, 0)),
            pl.BlockSpec((1, ROWS, 2 * NUM_EXPERTS),
                         lambda i: (i // TILES_PER_GROUP,
                                    i % TILES_PER_GROUP, 0)),
            pl.BlockSpec(block_shape=(1, 1, 1), index_map=lambda i: (i, 0, 0),
                         memory_space=pltpu.MemorySpace.SMEM),
        ],
        out_shape=[
            jax.ShapeDtypeStruct(paired, jnp.float32),
            jax.ShapeDtypeStruct(paired, jnp.float32),
            jax.ShapeDtypeStruct((n_tiles, 1, 1), jnp.float32),
        ],
        compiler_params=pltpu.CompilerParams(
            dimension_semantics=("parallel",),
        ),
    )(xp, W, b2)
    z_loss = jnp.sum(zparts) / n_tokens
    return (probs.reshape(shape3), logits.reshape(shape3), z_loss)


# speedup vs baseline: 4.5263x; 4.5263x over previous
import jax
import jax.numpy as jnp
from jax.experimental import pallas as pl
from jax.experimental.pallas import tpu as pltpu

NUM_GROUPS = 4
TOKENS_PER_GROUP = 8192
HIDDEN = 4096
NUM_EXPERTS = 64
TILE = 1024
TILES_PER_GROUP = TOKENS_PER_GROUP // TILE


def _router_body(x_ref, w_ref, b_ref, logits_ref, probs_ref, zpart_ref):
    x = x_ref[0]
    lt = jax.lax.dot_general(
        w_ref[...], x,
        dimension_numbers=(((0,), (1,)), ((), ())),
        preferred_element_type=jnp.float32,
    )
    lt = lt + b_ref[...]
    logits_ref[0] = lt
    m = jnp.max(lt, axis=0, keepdims=True)
    e = jnp.exp(lt - m)
    s = jnp.sum(e, axis=0, keepdims=True)
    probs_ref[0] = e / s
    log_z = m + jnp.log(s)
    zpart_ref[0, 0, 0] = jnp.sum(log_z * log_z)


def kernel(token_inputs, W, b, expert_capacity):
    n_tokens = NUM_GROUPS * TOKENS_PER_GROUP
    n_tiles = n_tokens // TILE
    bt = b.reshape(NUM_EXPERTS, 1)
    shape_t = (NUM_GROUPS, NUM_EXPERTS, TOKENS_PER_GROUP)
    logits_t, probs_t, zparts = pl.pallas_call(
        _router_body,
        grid=(n_tiles,),
        in_specs=[
            pl.BlockSpec((1, TILE, HIDDEN),
                         lambda i: (i // TILES_PER_GROUP,
                                    i % TILES_PER_GROUP, 0)),
            pl.BlockSpec((HIDDEN, NUM_EXPERTS), lambda i: (0, 0)),
            pl.BlockSpec((NUM_EXPERTS, 1), lambda i: (0, 0)),
        ],
        out_specs=[
            pl.BlockSpec((1, NUM_EXPERTS, TILE),
                         lambda i: (i // TILES_PER_GROUP, 0,
                                    i % TILES_PER_GROUP)),
            pl.BlockSpec((1, NUM_EXPERTS, TILE),
                         lambda i: (i // TILES_PER_GROUP, 0,
                                    i % TILES_PER_GROUP)),
            pl.BlockSpec(block_shape=(1, 1, 1), index_map=lambda i: (i, 0, 0),
                         memory_space=pltpu.MemorySpace.SMEM),
        ],
        out_shape=[
            jax.ShapeDtypeStruct(shape_t, jnp.float32),
            jax.ShapeDtypeStruct(shape_t, jnp.float32),
            jax.ShapeDtypeStruct((n_tiles, 1, 1), jnp.float32),
        ],
        compiler_params=pltpu.CompilerParams(
            dimension_semantics=("parallel",),
        ),
    )(token_inputs, W, bt)
    z_loss = jnp.sum(zparts) / n_tokens
    probs = jnp.transpose(probs_t, (0, 2, 1))
    logits = jnp.transpose(logits_t, (0, 2, 1))
    return (probs, logits, z_loss)


# speedup vs baseline: 4.6186x; 1.0204x over previous
import jax
import jax.numpy as jnp
from jax.experimental import pallas as pl
from jax.experimental.pallas import tpu as pltpu

NUM_GROUPS = 4
TOKENS_PER_GROUP = 8192
HIDDEN = 4096
NUM_EXPERTS = 64
TILE = 1024
TILES_PER_GROUP = TOKENS_PER_GROUP // TILE


def _router_body(x_ref, w_ref, b_ref, logits_ref, probs_ref, zpart_ref):
    x = x_ref[0]
    lt = jax.lax.dot_general(
        w_ref[...], x,
        dimension_numbers=(((1,), (1,)), ((), ())),
        preferred_element_type=jnp.float32,
    )
    lt = lt + b_ref[...]
    logits_ref[0] = lt
    m = jnp.max(lt, axis=0, keepdims=True)
    e = jnp.exp(lt - m)
    s = jnp.sum(e, axis=0, keepdims=True)
    probs_ref[0] = e / s
    log_z = m + jnp.log(s)
    zpart_ref[0, 0, 0] = jnp.sum(log_z * log_z)


def kernel(token_inputs, W, b, expert_capacity):
    n_tokens = NUM_GROUPS * TOKENS_PER_GROUP
    n_tiles = n_tokens // TILE
    wt = jnp.transpose(W)
    bt = b.reshape(NUM_EXPERTS, 1)
    shape_t = (NUM_GROUPS, NUM_EXPERTS, TOKENS_PER_GROUP)
    logits_t, probs_t, zparts = pl.pallas_call(
        _router_body,
        grid=(n_tiles,),
        in_specs=[
            pl.BlockSpec((1, TILE, HIDDEN),
                         lambda i: (i // TILES_PER_GROUP,
                                    i % TILES_PER_GROUP, 0)),
            pl.BlockSpec((NUM_EXPERTS, HIDDEN), lambda i: (0, 0)),
            pl.BlockSpec((NUM_EXPERTS, 1), lambda i: (0, 0)),
        ],
        out_specs=[
            pl.BlockSpec((1, NUM_EXPERTS, TILE),
                         lambda i: (i // TILES_PER_GROUP, 0,
                                    i % TILES_PER_GROUP)),
            pl.BlockSpec((1, NUM_EXPERTS, TILE),
                         lambda i: (i // TILES_PER_GROUP, 0,
                                    i % TILES_PER_GROUP)),
            pl.BlockSpec(block_shape=(1, 1, 1), index_map=lambda i: (i, 0, 0),
                         memory_space=pltpu.MemorySpace.SMEM),
        ],
        out_shape=[
            jax.ShapeDtypeStruct(shape_t, jnp.float32),
            jax.ShapeDtypeStruct(shape_t, jnp.float32),
            jax.ShapeDtypeStruct((n_tiles, 1, 1), jnp.float32),
        ],
        compiler_params=pltpu.CompilerParams(
            dimension_semantics=("parallel",),
        ),
    )(token_inputs, wt, bt)
    z_loss = jnp.sum(zparts) / n_tokens
    probs = jnp.transpose(probs_t, (0, 2, 1))
    logits = jnp.transpose(logits_t, (0, 2, 1))
    return (probs, logits, z_loss)


# speedup vs baseline: 4.6680x; 1.0107x over previous
import jax
import jax.numpy as jnp
from jax.experimental import pallas as pl
from jax.experimental.pallas import tpu as pltpu

NUM_GROUPS = 4
TOKENS_PER_GROUP = 8192
HIDDEN = 4096
NUM_EXPERTS = 64
TILE = 1024
TILES_PER_GROUP = TOKENS_PER_GROUP // TILE


def _router_body(x_ref, w_ref, b_ref, logits_ref, probs_ref, zpart_ref):
    x = x_ref[0]
    lt = jax.lax.dot_general(
        w_ref[...], x,
        dimension_numbers=(((1,), (1,)), ((), ())),
        preferred_element_type=jnp.float32,
    )
    lt = lt + b_ref[...]
    logits_ref[0] = lt
    m = jnp.max(lt, axis=0, keepdims=True)
    e = jnp.exp(lt - m)
    s = jnp.sum(e, axis=0, keepdims=True)
    probs_ref[0] = e / s
    log_z = m + jnp.log(s)
    part = jnp.sum(log_z * log_z)

    @pl.when(pl.program_id(0) == 0)
    def _():
        zpart_ref[0, 0] = 0.0

    zpart_ref[0, 0] += part


def kernel(token_inputs, W, b, expert_capacity):
    n_tokens = NUM_GROUPS * TOKENS_PER_GROUP
    n_tiles = n_tokens // TILE
    wt = jnp.transpose(W)
    bt = b.reshape(NUM_EXPERTS, 1)
    shape_t = (NUM_GROUPS, NUM_EXPERTS, TOKENS_PER_GROUP)
    logits_t, probs_t, zsum = pl.pallas_call(
        _router_body,
        grid=(n_tiles,),
        in_specs=[
            pl.BlockSpec((1, TILE, HIDDEN),
                         lambda i: (i // TILES_PER_GROUP,
                                    i % TILES_PER_GROUP, 0)),
            pl.BlockSpec((NUM_EXPERTS, HIDDEN), lambda i: (0, 0)),
            pl.BlockSpec((NUM_EXPERTS, 1), lambda i: (0, 0)),
        ],
        out_specs=[
            pl.BlockSpec((1, NUM_EXPERTS, TILE),
                         lambda i: (i // TILES_PER_GROUP, 0,
                                    i % TILES_PER_GROUP)),
            pl.BlockSpec((1, NUM_EXPERTS, TILE),
                         lambda i: (i // TILES_PER_GROUP, 0,
                                    i % TILES_PER_GROUP)),
            pl.BlockSpec(block_shape=(1, 1), index_map=lambda i: (0, 0),
                         memory_space=pltpu.MemorySpace.SMEM),
        ],
        out_shape=[
            jax.ShapeDtypeStruct(shape_t, jnp.float32),
            jax.ShapeDtypeStruct(shape_t, jnp.float32),
            jax.ShapeDtypeStruct((1, 1), jnp.float32),
        ],
        compiler_params=pltpu.CompilerParams(
            dimension_semantics=("arbitrary",),
        ),
    )(token_inputs, wt, bt)
    z_loss = zsum[0, 0] / n_tokens
    probs = jnp.transpose(probs_t, (0, 2, 1))
    logits = jnp.transpose(logits_t, (0, 2, 1))
    return (probs, logits, z_loss)
